# baseline (speedup 1.0000x reference)
import jax
import jax.numpy as jnp
from jax.experimental import pallas as pl
from jax.experimental.pallas import tpu as pltpu

M = 256
H = 2880
I = 2880
BK = 480
NK = H // BK
BJ = 480
NJ = I // BJ
OSS_ALPHA = 1.702
OSS_LIMIT = 7.0

_DN0 = (((0,), (0,)), ((), ()))


def _mlp_body(x_ref, gw_ref, uw_ref, gb_ref, ub_ref, dw_ref, db_ref,
              out_ref, xt_ref, g_ref, u_ref, ht_ref):
    s = pl.program_id(0)

    @pl.when(s == 0)
    def _xpose():
        xt_ref[...] = x_ref[...].T

    @pl.when(jnp.logical_and(s >= 1, s <= NK))
    def _phase1():
        k = s - 1
        xt = xt_ref[pl.ds(k * BK, BK), :]
        gp = jax.lax.dot_general(xt, gw_ref[...], _DN0,
                                 preferred_element_type=jnp.float32)
        up = jax.lax.dot_general(xt, uw_ref[...], _DN0,
                                 preferred_element_type=jnp.float32)

        @pl.when(s == 1)
        def _init():
            g_ref[...] = gp
            u_ref[...] = up

        @pl.when(s > 1)
        def _accum():
            g_ref[...] += gp
            u_ref[...] += up

        @pl.when(s == NK)
        def _finish():
            g = g_ref[...] + gb_ref[...]
            u = u_ref[...] + ub_ref[...]
            u = jnp.clip(u, -OSS_LIMIT, OSS_LIMIT)
            g = jnp.minimum(g, OSS_LIMIT)
            glu = g * (1.0 / (1.0 + jnp.exp(-OSS_ALPHA * g)))
            ht_ref[...] = (glu * (u + 1.0)).T

    @pl.when(s > NK)
    def _phase2():
        j = s - NK - 1
        ht_blk = ht_ref[pl.ds(j * BJ, BJ), :]
        acc = jax.lax.dot_general(ht_blk, dw_ref[...], _DN0,
                                  preferred_element_type=jnp.float32)

        @pl.when(s == NK + 1)
        def _init():
            out_ref[...] = acc + db_ref[...]

        @pl.when(s > NK + 1)
        def _accum():
            out_ref[...] += acc


def kernel(x, gate_w, gate_b, up_w, up_b, down_w, down_b):
    return pl.pallas_call(
        _mlp_body,
        grid=(1 + NK + NJ,),
        in_specs=[
            pl.BlockSpec((M, H), lambda s: (0, 0)),
            pl.BlockSpec((BK, I), lambda s: (jnp.clip(s - 1, 0, NK - 1), 0)),
            pl.BlockSpec((BK, I), lambda s: (jnp.clip(s - 1, 0, NK - 1), 0)),
            pl.BlockSpec((1, I), lambda s: (0, 0)),
            pl.BlockSpec((1, I), lambda s: (0, 0)),
            pl.BlockSpec((BJ, H),
                         lambda s: (jnp.clip(s - NK - 1, 0, NJ - 1), 0)),
            pl.BlockSpec((1, H), lambda s: (0, 0)),
        ],
        out_specs=pl.BlockSpec((M, H), lambda s: (0, 0)),
        out_shape=jax.ShapeDtypeStruct((M, H), jnp.float32),
        scratch_shapes=[
            pltpu.VMEM((H, M), jnp.float32),
            pltpu.VMEM((M, I), jnp.float32),
            pltpu.VMEM((M, I), jnp.float32),
            pltpu.VMEM((I, M), jnp.float32),
        ],
    )(x, gate_w, up_w, gate_b, up_b, down_w, down_b)


# speedup vs baseline: 1.0881x; 1.0881x over previous
import jax
import jax.numpy as jnp
from jax.experimental import pallas as pl
from jax.experimental.pallas import tpu as pltpu

M = 256
H = 2880
I = 2880
BK = 480
NK = H // BK
BJ = 720
NJ = I // BJ
OSS_ALPHA = 1.702
OSS_LIMIT = 7.0

_DN0 = (((0,), (0,)), ((), ()))


def _mlp_body(xt_ref, gw_ref, uw_ref, gb_ref, ub_ref, dw_ref, db_ref,
              out_ref, g_ref, u_ref, ht_ref):
    s = pl.program_id(0)

    @pl.when(s < NK)
    def _phase1():
        xt = xt_ref[...]
        gp = jax.lax.dot_general(xt, gw_ref[...], _DN0,
                                 preferred_element_type=jnp.float32)
        up = jax.lax.dot_general(xt, uw_ref[...], _DN0,
                                 preferred_element_type=jnp.float32)

        @pl.when(s == 0)
        def _init():
            g_ref[...] = gp
            u_ref[...] = up

        @pl.when(s > 0)
        def _accum():
            g_ref[...] += gp
            u_ref[...] += up

        @pl.when(s == NK - 1)
        def _finish():
            g = g_ref[...] + gb_ref[...]
            u = u_ref[...] + ub_ref[...]
            u = jnp.clip(u, -OSS_LIMIT, OSS_LIMIT)
            g = jnp.minimum(g, OSS_LIMIT)
            glu = g * (1.0 / (1.0 + jnp.exp(-OSS_ALPHA * g)))
            ht_ref[...] = (glu * (u + 1.0)).astype(jnp.bfloat16).T

    @pl.when(s >= NK)
    def _phase2():
        j = s - NK
        ht_blk = ht_ref[pl.ds(j * BJ, BJ), :]
        acc = jax.lax.dot_general(ht_blk, dw_ref[...], _DN0,
                                  preferred_element_type=jnp.float32)

        @pl.when(s == NK)
        def _init():
            out_ref[...] = acc + db_ref[...]

        @pl.when(s > NK)
        def _accum():
            out_ref[...] += acc


def kernel(x, gate_w, gate_b, up_w, up_b, down_w, down_b):
    xt = x.T.astype(jnp.bfloat16)
    return pl.pallas_call(
        _mlp_body,
        grid=(NK + NJ,),
        in_specs=[
            pl.BlockSpec((BK, M), lambda s: (jnp.minimum(s, NK - 1), 0)),
            pl.BlockSpec((BK, I), lambda s: (jnp.minimum(s, NK - 1), 0)),
            pl.BlockSpec((BK, I), lambda s: (jnp.minimum(s, NK - 1), 0)),
            pl.BlockSpec((1, I), lambda s: (0, 0)),
            pl.BlockSpec((1, I), lambda s: (0, 0)),
            pl.BlockSpec((BJ, H),
                         lambda s: (jnp.clip(s - NK, 0, NJ - 1), 0)),
            pl.BlockSpec((1, H), lambda s: (0, 0)),
        ],
        out_specs=pl.BlockSpec((M, H), lambda s: (0, 0)),
        out_shape=jax.ShapeDtypeStruct((M, H), jnp.float32),
        scratch_shapes=[
            pltpu.VMEM((M, I), jnp.float32),
            pltpu.VMEM((M, I), jnp.float32),
            pltpu.VMEM((I, M), jnp.bfloat16),
        ],
    )(xt, gate_w, up_w, gate_b, up_b, down_w, down_b)


# speedup vs baseline: 1.1745x; 1.0794x over previous
import jax
import jax.numpy as jnp
from jax.experimental import pallas as pl
from jax.experimental.pallas import tpu as pltpu

M = 256
H = 2880
I = 2880
BK = 480
NK = H // BK
BJ = 720
NJ = I // BJ
OSS_ALPHA = 1.702
OSS_LIMIT = 7.0

_DN0 = (((0,), (0,)), ((), ()))


def _mlp_body(xt_ref, gw_ref, uw_ref, gb_ref, ub_ref, dw_ref, db_ref,
              out_ref, g_ref, u_ref, ht_ref):
    s = pl.program_id(0)

    @pl.when(s < NK)
    def _phase1():
        xt = xt_ref[...]
        gp = jax.lax.dot_general(xt, gw_ref[...], _DN0,
                                 preferred_element_type=jnp.float32)
        up = jax.lax.dot_general(xt, uw_ref[...], _DN0,
                                 preferred_element_type=jnp.float32)

        @pl.when(s == 0)
        def _init():
            g_ref[...] = gp
            u_ref[...] = up

        @pl.when(s > 0)
        def _accum():
            g_ref[...] += gp
            u_ref[...] += up

        @pl.when(s == NK - 1)
        def _finish():
            g = g_ref[...] + gb_ref[...]
            u = u_ref[...] + ub_ref[...]
            u = jnp.clip(u, -OSS_LIMIT, OSS_LIMIT)
            g = jnp.minimum(g, OSS_LIMIT)
            glu = g * (1.0 / (1.0 + jnp.exp(-OSS_ALPHA * g)))
            ht_ref[...] = (glu * (u + 1.0)).T

    @pl.when(s >= NK)
    def _phase2():
        j = s - NK
        ht_blk = ht_ref[pl.ds(j * BJ, BJ), :]
        acc = jax.lax.dot_general(ht_blk, dw_ref[...], _DN0,
                                  preferred_element_type=jnp.float32)

        @pl.when(s == NK)
        def _init():
            out_ref[...] = acc + db_ref[...]

        @pl.when(s > NK)
        def _accum():
            out_ref[...] += acc


def kernel(x, gate_w, gate_b, up_w, up_b, down_w, down_b):
    xt = x.T
    return pl.pallas_call(
        _mlp_body,
        grid=(NK + NJ,),
        in_specs=[
            pl.BlockSpec((BK, M), lambda s: (jnp.minimum(s, NK - 1), 0)),
            pl.BlockSpec((BK, I), lambda s: (jnp.minimum(s, NK - 1), 0)),
            pl.BlockSpec((BK, I), lambda s: (jnp.minimum(s, NK - 1), 0)),
            pl.BlockSpec((1, I), lambda s: (0, 0)),
            pl.BlockSpec((1, I), lambda s: (0, 0)),
            pl.BlockSpec((BJ, H),
                         lambda s: (jnp.clip(s - NK, 0, NJ - 1), 0)),
            pl.BlockSpec((1, H), lambda s: (0, 0)),
        ],
        out_specs=pl.BlockSpec((M, H), lambda s: (0, 0)),
        out_shape=jax.ShapeDtypeStruct((M, H), jnp.float32),
        scratch_shapes=[
            pltpu.VMEM((M, I), jnp.float32),
            pltpu.VMEM((M, I), jnp.float32),
            pltpu.VMEM((I, M), jnp.float32),
        ],
    )(xt, gate_w, up_w, gate_b, up_b, down_w, down_b)
